# baseline (speedup 1.0000x reference)
import jax
import jax.numpy as jnp
from jax import lax
from jax.experimental import pallas as pl
from jax.experimental.pallas import tpu as pltpu
from jax.experimental.pallas import tpu_sc as plsc

BATCH = 8192
DIM = 2048
NC = 2
NS = 16
NW = NC * NS
LANES = 16
ROWS_PER_W = BATCH // NW
R = 4
CHUNKS = ROWS_PER_W // R
NBUF = 4
JGROUPS = DIM // LANES


def _body(in_hbm, perm_hbm, out_hbm, ld_hbm, perm_v, zero_v, spm, *bufs):
    in_v = bufs[0:NBUF]
    out_v = bufs[NBUF:2 * NBUF]
    si = bufs[2 * NBUF:3 * NBUF]
    so = bufs[3 * NBUF:4 * NBUF]
    st = bufs[4 * NBUF:4 * NBUF + 2]

    sid = lax.axis_index("s")
    wid = sid * NC + lax.axis_index("c")
    row0 = wid * ROWS_PER_W

    def in_cp(c, b):
        return pltpu.make_async_copy(
            in_hbm.at[pl.ds(row0 + c * R, R), :], in_v[b], si[b])

    def out_cp(c, b):
        if b % 2 == 0:
            return pltpu.make_async_copy(
                out_v[b], out_hbm.at[pl.ds(row0 + c * R, R), :], so[b])
        return pltpu.make_async_copy(
            spm.at[sid, b // 2], out_hbm.at[pl.ds(row0 + c * R, R), :], so[b])

    def stage_cp(b):
        return pltpu.make_async_copy(out_v[b], spm.at[sid, b // 2], st[b // 2])

    for b in range(NBUF):
        in_cp(b, b).start()

    for i in range(ROWS_PER_W // LANES):
        zero_v[pl.ds(i * LANES, LANES)] = jnp.zeros((LANES,), jnp.float32)
    pltpu.sync_copy(zero_v, ld_hbm.at[pl.ds(wid * ROWS_PER_W, ROWS_PER_W)])

    pltpu.sync_copy(perm_hbm, perm_v)

    def gather_chunk(inbuf, outbuf):
        @plsc.parallel_loop(0, JGROUPS, unroll=4)
        def jg_body(jg):
            idx = perm_v[pl.ds(jg * LANES, LANES)]
            for r in range(R):
                row_idx = jnp.full((LANES,), r, jnp.int32)
                g = plsc.load_gather(inbuf, [row_idx, idx])
                outbuf[r, pl.ds(jg * LANES, LANES)] = g

    def super_body(k, carry):
        for b in range(NBUF):
            c = k * NBUF + b
            in_cp(c, b).wait()

            @pl.when(k > 0)
            def _wait_out():
                out_cp(c - NBUF, b).wait()

            gather_chunk(in_v[b], out_v[b])
            if b % 2 == 1:
                stage_cp(b).start()
                stage_cp(b).wait()
            out_cp(c, b).start()

            @pl.when(k < (CHUNKS // NBUF - 1))
            def _start_next_in():
                in_cp(c + NBUF, b).start()
        return carry

    lax.fori_loop(0, CHUNKS // NBUF, super_body, None)
    for b in range(NBUF):
        out_cp(CHUNKS - NBUF + b, b).wait()


@jax.jit
def kernel(inputs, perm):
    permute = pl.kernel(
        _body,
        out_type=[
            jax.ShapeDtypeStruct((BATCH, DIM), jnp.float32),
            jax.ShapeDtypeStruct((BATCH,), jnp.float32),
        ],
        mesh=plsc.VectorSubcoreMesh(core_axis_name="c", subcore_axis_name="s"),
        compiler_params=pltpu.CompilerParams(needs_layout_passes=False),
        scratch_types=(
            [pltpu.VMEM((DIM,), jnp.int32),
             pltpu.VMEM((ROWS_PER_W,), jnp.float32),
             pltpu.VMEM_SHARED((NS, 2, R, DIM), jnp.float32)]
            + [pltpu.VMEM((R, DIM), jnp.float32) for _ in range(2 * NBUF)]
            + [pltpu.SemaphoreType.DMA for _ in range(2 * NBUF + 2)]
        ),
    )
    out, logdet = permute(inputs, perm.astype(jnp.int32))
    return (out, logdet)


# speedup vs baseline: 1.0276x; 1.0276x over previous
import jax
import jax.numpy as jnp
from jax import lax
from jax.experimental import pallas as pl
from jax.experimental.pallas import tpu as pltpu
from jax.experimental.pallas import tpu_sc as plsc

BATCH = 8192
DIM = 2048
NC = 2
NS = 16
NW = NC * NS
LANES = 16
ROWS_PER_W = BATCH // NW
R = 4
CHUNKS = ROWS_PER_W // R
NBUF = 4
JGROUPS = DIM // LANES


def _body(in_hbm, perm_hbm, out_hbm, ld_hbm, perm_v, zero_v, *bufs):
    in_v = bufs[0:NBUF]
    out_v = bufs[NBUF:2 * NBUF]
    si = bufs[2 * NBUF:3 * NBUF]
    so = bufs[3 * NBUF:4 * NBUF]

    wid = lax.axis_index("s") * NC + lax.axis_index("c")
    row0 = wid * ROWS_PER_W

    def in_cp(c, b):
        return pltpu.make_async_copy(
            in_hbm.at[pl.ds(row0 + c * R, R), :], in_v[b], si[b])

    def out_cp(c, b):
        return pltpu.make_async_copy(
            out_v[b], out_hbm.at[pl.ds(row0 + c * R, R), :], so[b])

    for b in range(NBUF):
        in_cp(b, b).start()

    for i in range(ROWS_PER_W // LANES):
        zero_v[pl.ds(i * LANES, LANES)] = jnp.zeros((LANES,), jnp.float32)
    pltpu.sync_copy(zero_v, ld_hbm.at[pl.ds(wid * ROWS_PER_W, ROWS_PER_W)])

    pltpu.sync_copy(perm_hbm, perm_v)

    def gather_chunk(inbuf, outbuf):
        @plsc.parallel_loop(0, JGROUPS, unroll=4)
        def jg_body(jg):
            idx = perm_v[pl.ds(jg * LANES, LANES)]
            for r in range(R):
                row_idx = jnp.full((LANES,), r, jnp.int32)
                g = plsc.load_gather(inbuf, [row_idx, idx])
                outbuf[r, pl.ds(jg * LANES, LANES)] = g

    def super_body(k, carry):
        for b in range(NBUF):
            c = k * NBUF + b
            in_cp(c, b).wait()

            @pl.when(k > 0)
            def _wait_out():
                out_cp(c - NBUF, b).wait()

            gather_chunk(in_v[b], out_v[b])
            out_cp(c, b).start()

            @pl.when(k < (CHUNKS // NBUF - 1))
            def _start_next_in():
                in_cp(c + NBUF, b).start()
        return carry

    lax.fori_loop(0, CHUNKS // NBUF, super_body, None)
    for b in range(NBUF):
        out_cp(CHUNKS - NBUF + b, b).wait()


@jax.jit
def kernel(inputs, perm):
    permute = pl.kernel(
        _body,
        out_type=[
            jax.ShapeDtypeStruct((BATCH, DIM), jnp.float32),
            jax.ShapeDtypeStruct((BATCH,), jnp.float32),
        ],
        mesh=plsc.VectorSubcoreMesh(core_axis_name="c", subcore_axis_name="s"),
        compiler_params=pltpu.CompilerParams(needs_layout_passes=False),
        scratch_types=(
            [pltpu.VMEM((DIM,), jnp.int32),
             pltpu.VMEM((ROWS_PER_W,), jnp.float32)]
            + [pltpu.VMEM((R, DIM), jnp.float32) for _ in range(2 * NBUF)]
            + [pltpu.SemaphoreType.DMA for _ in range(2 * NBUF)]
        ),
    )
    out, logdet = permute(inputs, perm.astype(jnp.int32))
    return (out, logdet)
